# baseline (speedup 1.0000x reference)
import functools

import jax
import jax.numpy as jnp
from jax import lax
from jax.experimental import pallas as pl
from jax.experimental.pallas import tpu as pltpu
from jax.experimental.pallas import tpu_sc as plsc

BATCH = 16384
DIM = 64
H1 = 64
H2 = 32
NC = 2
NS = 16
NW = NC * NS
BPW = BATCH // NW


def _sc_gather(user, item, user_emb, item_emb):
    mesh = plsc.VectorSubcoreMesh(core_axis_name="c", subcore_axis_name="s")

    @functools.partial(
        pl.kernel,
        mesh=mesh,
        compiler_params=pltpu.CompilerParams(use_tc_tiling_on_sc=False),
        out_type=(
            jax.ShapeDtypeStruct((BATCH, DIM), jnp.float32),
            jax.ShapeDtypeStruct((BATCH, DIM), jnp.float32),
        ),
        scratch_types=[
            pltpu.VMEM((BPW,), jnp.int32),
            pltpu.VMEM((BPW,), jnp.int32),
            pltpu.VMEM((BPW, DIM), jnp.float32),
            pltpu.VMEM((BPW, DIM), jnp.float32),
            pltpu.SemaphoreType.DMA,
            pltpu.SemaphoreType.DMA,
        ],
    )
    def k(user_hbm, item_hbm, uemb_hbm, iemb_hbm, uout_hbm, iout_hbm,
          uidx_v, iidx_v, urows_v, irows_v, usem, isem):
        wid = lax.axis_index("s") * NC + lax.axis_index("c")
        base = wid * BPW
        pltpu.sync_copy(user_hbm.at[pl.ds(base, BPW)], uidx_v)
        pltpu.sync_copy(item_hbm.at[pl.ds(base, BPW)], iidx_v)
        ucp = pltpu.async_copy(uemb_hbm.at[uidx_v], urows_v, usem)
        icp = pltpu.async_copy(iemb_hbm.at[iidx_v], irows_v, isem)
        ucp.wait()
        pltpu.sync_copy(urows_v, uout_hbm.at[pl.ds(base, BPW)])
        icp.wait()
        pltpu.sync_copy(irows_v, iout_hbm.at[pl.ds(base, BPW)])

    return k(user, item, user_emb, item_emb)


def _tc_mlp(uf, itf, W1, b1, W2, b2):
    BLK = 2048

    def body(uf_ref, if_ref, w1u_ref, w1i_ref, b1_ref, w2_ref, b2_ref, out_ref):
        a = (jnp.dot(uf_ref[...], w1u_ref[...], preferred_element_type=jnp.float32)
             + jnp.dot(if_ref[...], w1i_ref[...], preferred_element_type=jnp.float32)
             + b1_ref[...])
        out = jnp.dot(a, w2_ref[...], preferred_element_type=jnp.float32) + b2_ref[...]
        out_ref[...] = jnp.maximum(out, 0.0)

    return pl.pallas_call(
        body,
        grid=(BATCH // BLK,),
        in_specs=[
            pl.BlockSpec((BLK, DIM), lambda i: (i, 0)),
            pl.BlockSpec((BLK, DIM), lambda i: (i, 0)),
            pl.BlockSpec((DIM, H1), lambda i: (0, 0)),
            pl.BlockSpec((DIM, H1), lambda i: (0, 0)),
            pl.BlockSpec((1, H1), lambda i: (0, 0)),
            pl.BlockSpec((H1, H2), lambda i: (0, 0)),
            pl.BlockSpec((1, H2), lambda i: (0, 0)),
        ],
        out_specs=pl.BlockSpec((BLK, H2), lambda i: (i, 0)),
        out_shape=jax.ShapeDtypeStruct((BATCH, H2), jnp.float32),
    )(uf, itf, W1[:DIM], W1[DIM:], b1.reshape(1, H1), W2, b2.reshape(1, H2))


def kernel(user, item, user_emb, item_emb, W1, b1, W2, b2):
    uf, itf = _sc_gather(user, item, user_emb, item_emb)
    return _tc_mlp(uf, itf, W1, b1, W2, b2)


# speedup vs baseline: 1.5811x; 1.5811x over previous
import functools

import jax
import jax.numpy as jnp
from jax import lax
from jax.experimental import pallas as pl
from jax.experimental.pallas import tpu as pltpu
from jax.experimental.pallas import tpu_sc as plsc

BATCH = 16384
DIM = 64
H1 = 64
H2 = 32
NC = 2
NS = 16
NW = NC * NS
BPW = BATCH // NW
CH = 128
NCH = BPW // CH


def _sc_gather_join(user, item, user_emb, item_emb):
    mesh = plsc.VectorSubcoreMesh(core_axis_name="c", subcore_axis_name="s")

    @functools.partial(
        pl.kernel,
        mesh=mesh,
        out_type=jax.ShapeDtypeStruct((BATCH, 2 * DIM), jnp.float32),
        scratch_types=[
            pltpu.VMEM((BPW,), jnp.int32),
            pltpu.VMEM((BPW,), jnp.int32),
            pltpu.VMEM((2, CH, 2 * DIM), jnp.float32),
            pltpu.SemaphoreType.DMA,
            pltpu.SemaphoreType.DMA,
        ],
    )
    def k(user_hbm, item_hbm, uemb_hbm, iemb_hbm, joined_hbm,
          uidx_v, iidx_v, join_v, gsem, wsem):
        wid = lax.axis_index("s") * NC + lax.axis_index("c")
        base = wid * BPW
        pltpu.sync_copy(user_hbm.at[pl.ds(base, BPW)], uidx_v)
        pltpu.sync_copy(item_hbm.at[pl.ds(base, BPW)], iidx_v)

        wbs = {}
        for c in range(NCH):
            buf = c % 2
            if c >= 2:
                wbs.pop(c - 2).wait()

            def issue(g, _, c=c, buf=buf):
                uv = uidx_v[pl.ds(c * CH + g * 16, 16)]
                iv = iidx_v[pl.ds(c * CH + g * 16, 16)]
                for l in range(16):
                    j = g * 16 + l
                    pltpu.async_copy(
                        uemb_hbm.at[uv[l]],
                        join_v.at[buf, j, pl.ds(0, DIM)], gsem)
                    pltpu.async_copy(
                        iemb_hbm.at[iv[l]],
                        join_v.at[buf, j, pl.ds(DIM, DIM)], gsem)
                return _

            lax.fori_loop(0, CH // 16, issue, 0)
            pltpu.make_async_copy(
                joined_hbm.at[pl.ds(0, CH)], join_v.at[buf], gsem).wait()
            wbs[c] = pltpu.async_copy(
                join_v.at[buf], joined_hbm.at[pl.ds(base + c * CH, CH)], wsem)
        for c in sorted(wbs):
            wbs.pop(c).wait()

    return k(user, item, user_emb, item_emb)


def _tc_mlp(joined, W1, b1, W2, b2):
    BLK = 2048

    def body(j_ref, w1_ref, b1_ref, w2_ref, b2_ref, out_ref):
        a = (jnp.dot(j_ref[...], w1_ref[...], preferred_element_type=jnp.float32)
             + b1_ref[...])
        out = jnp.dot(a, w2_ref[...], preferred_element_type=jnp.float32) + b2_ref[...]
        out_ref[...] = jnp.maximum(out, 0.0)

    return pl.pallas_call(
        body,
        grid=(BATCH // BLK,),
        in_specs=[
            pl.BlockSpec((BLK, 2 * DIM), lambda i: (i, 0)),
            pl.BlockSpec((2 * DIM, H1), lambda i: (0, 0)),
            pl.BlockSpec((1, H1), lambda i: (0, 0)),
            pl.BlockSpec((H1, H2), lambda i: (0, 0)),
            pl.BlockSpec((1, H2), lambda i: (0, 0)),
        ],
        out_specs=pl.BlockSpec((BLK, H2), lambda i: (i, 0)),
        out_shape=jax.ShapeDtypeStruct((BATCH, H2), jnp.float32),
    )(joined, W1, b1.reshape(1, H1), W2, b2.reshape(1, H2))


def kernel(user, item, user_emb, item_emb, W1, b1, W2, b2):
    joined = _sc_gather_join(user, item, user_emb, item_emb)
    return _tc_mlp(joined, W1, b1, W2, b2)


# speedup vs baseline: 1.6380x; 1.0360x over previous
import functools

import jax
import jax.numpy as jnp
from jax import lax
from jax.experimental import pallas as pl
from jax.experimental.pallas import tpu as pltpu
from jax.experimental.pallas import tpu_sc as plsc

BATCH = 16384
VOC = 1000000
DIM = 64
H1 = 64
H2 = 32
NC = 2
NS = 16
NW = NC * NS
BPW = BATCH // NW
CH = 128
NCH = BPW // CH
BLKP = 2048


def _tc_transform(uemb_t, iemb_t, W1, W2):
    def body(ut_ref, it_ref, w1_ref, w2_ref, out_ref):
        w12u = jnp.dot(w1_ref[0:DIM, :], w2_ref[...],
                       preferred_element_type=jnp.float32)
        w12i = jnp.dot(w1_ref[DIM:2 * DIM, :], w2_ref[...],
                       preferred_element_type=jnp.float32)
        tu = lax.dot_general(ut_ref[...], w12u, (((0,), (0,)), ((), ())),
                             preferred_element_type=jnp.float32)
        ti = lax.dot_general(it_ref[...], w12i, (((0,), (0,)), ((), ())),
                             preferred_element_type=jnp.float32)
        out_ref[...] = jnp.concatenate([tu, ti], axis=1)

    grid = (pl.cdiv(VOC, BLKP),)
    return pl.pallas_call(
        body,
        grid=grid,
        in_specs=[
            pl.BlockSpec((DIM, BLKP), lambda i: (0, i)),
            pl.BlockSpec((DIM, BLKP), lambda i: (0, i)),
            pl.BlockSpec((2 * DIM, H1), lambda i: (0, 0)),
            pl.BlockSpec((H1, H2), lambda i: (0, 0)),
        ],
        out_specs=pl.BlockSpec((BLKP, 2 * H2), lambda i: (i, 0)),
        out_shape=jax.ShapeDtypeStruct((VOC, 2 * H2), jnp.float32),
    )(uemb_t, iemb_t, W1, W2)


def _sc_gather_join(user, item, table):
    mesh = plsc.VectorSubcoreMesh(core_axis_name="c", subcore_axis_name="s")

    @functools.partial(
        pl.kernel,
        mesh=mesh,
        out_type=jax.ShapeDtypeStruct((BATCH, 2 * DIM), jnp.float32),
        scratch_types=[
            pltpu.VMEM((BPW,), jnp.int32),
            pltpu.VMEM((BPW,), jnp.int32),
            pltpu.VMEM((2, CH, 2 * DIM), jnp.float32),
            pltpu.SemaphoreType.DMA,
            pltpu.SemaphoreType.DMA,
        ],
    )
    def k(user_hbm, item_hbm, t_hbm, joined_hbm,
          uidx_v, iidx_v, join_v, gsem, wsem):
        wid = lax.axis_index("s") * NC + lax.axis_index("c")
        base = wid * BPW
        pltpu.sync_copy(user_hbm.at[pl.ds(base, BPW)], uidx_v)
        pltpu.sync_copy(item_hbm.at[pl.ds(base, BPW)], iidx_v)

        wbs = {}
        for c in range(NCH):
            buf = c % 2
            if c >= 2:
                wbs.pop(c - 2).wait()

            def issue(g, _, c=c, buf=buf):
                uv = uidx_v[pl.ds(c * CH + g * 16, 16)]
                iv = iidx_v[pl.ds(c * CH + g * 16, 16)]
                for l in range(16):
                    j = g * 16 + l
                    pltpu.async_copy(
                        t_hbm.at[uv[l]],
                        join_v.at[buf, j, pl.ds(0, DIM)], gsem)
                    pltpu.async_copy(
                        t_hbm.at[iv[l]],
                        join_v.at[buf, j, pl.ds(DIM, DIM)], gsem)
                return _

            lax.fori_loop(0, CH // 16, issue, 0)
            pltpu.make_async_copy(
                joined_hbm.at[pl.ds(0, CH)], join_v.at[buf], gsem).wait()
            wbs[c] = pltpu.async_copy(
                join_v.at[buf], joined_hbm.at[pl.ds(base + c * CH, CH)], wsem)
        for c in sorted(wbs):
            wbs.pop(c).wait()

    return k(user, item, table)


def _tc_finish(joined, W2, b1, b2):
    BLK = 4096

    def body(j_ref, w2_ref, b1_ref, b2_ref, out_ref):
        bias = jnp.dot(b1_ref[...], w2_ref[...],
                       preferred_element_type=jnp.float32) + b2_ref[...]
        s = j_ref[:, 0:H2] + j_ref[:, 3 * H2:4 * H2] + bias
        out_ref[...] = jnp.maximum(s, 0.0)

    return pl.pallas_call(
        body,
        grid=(BATCH // BLK,),
        in_specs=[
            pl.BlockSpec((BLK, 2 * DIM), lambda i: (i, 0)),
            pl.BlockSpec((H1, H2), lambda i: (0, 0)),
            pl.BlockSpec((1, H1), lambda i: (0, 0)),
            pl.BlockSpec((1, H2), lambda i: (0, 0)),
        ],
        out_specs=pl.BlockSpec((BLK, H2), lambda i: (i, 0)),
        out_shape=jax.ShapeDtypeStruct((BATCH, H2), jnp.float32),
    )(joined, W2, b1.reshape(1, H1), b2.reshape(1, H2))


def kernel(user, item, user_emb, item_emb, W1, b1, W2, b2):
    t = _tc_transform(user_emb.T, item_emb.T, W1, W2)
    joined = _sc_gather_join(user, item, t)
    return _tc_finish(joined, W2, b1, b2)
